# Initial kernel scaffold; baseline (speedup 1.0000x reference)
import jax
import jax.numpy as jnp
from jax.experimental import pallas as pl


def kernel(x, W_learn, W_frozen):
    raise NotImplementedError("write your pallas kernel here")


# speedup vs baseline: 5.5861x; 5.5861x over previous
import numpy as _np

import jax
import jax.numpy as jnp
from jax import lax
from jax.experimental import pallas as pl
from jax.experimental.pallas import tpu as pltpu
from jax.experimental.pallas import tpu_sc as plsc

_N = 16384 * 200
_NW = 32
_PER_W = _N // _NW
_CHUNK = 4096
_N_CHUNKS = _PER_W // _CHUNK
_GROUPS = _CHUNK // 16
_L = 16


def _body(x_hbm, tab_hbm, const_hbm, out_l_hbm, out_f_hbm,
          tab_v, const_v, idx_v, outl_v, outf_v):
    nc = 2
    wid = lax.axis_index("s") * nc + lax.axis_index("c")
    base_w = wid * _PER_W

    pltpu.sync_copy(tab_hbm, tab_v)
    pltpu.sync_copy(const_hbm, const_v)

    perms = [const_v[pl.ds(k * _L, _L)] for k in range(5)]
    cols_l = [const_v[pl.ds((5 + k) * _L, _L)] for k in range(5)]
    cols_f = [const_v[pl.ds((10 + k) * _L, _L)] for k in range(5)]

    def chunk_body(c, _):
        start = base_w + c * _CHUNK
        pltpu.sync_copy(x_hbm.at[pl.ds(start, _CHUNK)], idx_v)

        def group_body(g, _):
            gbase = jnp.full((_L,), g * _L, jnp.int32)
            obase = g * (5 * _L)
            for k in range(5):
                rows = plsc.load_gather(idx_v, [gbase + perms[k]])
                r5 = rows * 5
                val_l = plsc.load_gather(tab_v, [r5 + cols_l[k]])
                val_f = plsc.load_gather(tab_v, [r5 + cols_f[k]])
                outl_v[pl.ds(obase + k * _L, _L)] = val_l
                outf_v[pl.ds(obase + k * _L, _L)] = val_f
            return 0

        lax.fori_loop(0, _GROUPS, group_body, 0)
        pltpu.sync_copy(outl_v, out_l_hbm.at[pl.ds(start * 5, _CHUNK * 5)])
        pltpu.sync_copy(outf_v, out_f_hbm.at[pl.ds(start * 5, _CHUNK * 5)])
        return 0

    lax.fori_loop(0, _N_CHUNKS, chunk_body, 0)


def _expand_consts():
    j = _np.arange(_L)
    rows = [(k * _L + j) // 5 for k in range(5)]
    cols_l = [(k * _L + j) % 5 for k in range(5)]
    cols_f = [c + 50 for c in cols_l]
    return _np.concatenate(rows + cols_l + cols_f).astype(_np.int32)


_CONSTS = _expand_consts()


@jax.jit
def _run(x_flat, tab):
    mesh = plsc.VectorSubcoreMesh(core_axis_name="c", subcore_axis_name="s")
    f = pl.kernel(
        _body,
        out_type=(
            jax.ShapeDtypeStruct((_N * 5,), jnp.float32),
            jax.ShapeDtypeStruct((_N * 5,), jnp.float32),
        ),
        mesh=mesh,
        compiler_params=pltpu.CompilerParams(needs_layout_passes=False),
        scratch_types=[
            pltpu.VMEM((128,), jnp.float32),
            pltpu.VMEM((240,), jnp.int32),
            pltpu.VMEM((_CHUNK,), jnp.int32),
            pltpu.VMEM((_CHUNK * 5,), jnp.float32),
            pltpu.VMEM((_CHUNK * 5,), jnp.float32),
        ],
    )
    return f(x_flat, tab, jnp.asarray(_CONSTS))


def kernel(x, W_learn, W_frozen):
    x_flat = x.reshape(-1).astype(jnp.int32)
    tab = jnp.concatenate(
        [W_learn.reshape(-1), W_frozen.reshape(-1),
         jnp.zeros((28,), jnp.float32)]
    )
    out_l, out_f = _run(x_flat, tab)
    shp = x.shape + (5,)
    return (out_l.reshape(shp), out_f.reshape(shp))


# speedup vs baseline: 73.0591x; 13.0788x over previous
import numpy as _np

import jax
import jax.numpy as jnp
from jax import lax
from jax.experimental import pallas as pl
from jax.experimental.pallas import tpu as pltpu
from jax.experimental.pallas import tpu_sc as plsc

_B = 16384
_S = 200
_N = _B * _S
_NW = 32
_BW = _B // _NW
_NCH = _S // 8
_L = 16


def _body(xt_hbm, tab_hbm, out_l_hbm, out_f_hbm, tab_v, idx_v, outl_v, outf_v):
    wid = lax.axis_index("s") * 2 + lax.axis_index("c")
    bcol = wid * _BW

    pltpu.sync_copy(tab_hbm, tab_v)

    def chunk(i, _):
        pltpu.sync_copy(xt_hbm.at[pl.ds(i * 8, 8), pl.ds(bcol, _BW)], idx_v)

        def row_body(s_in, _):
            def grp_body(bb, _):
                for bt in range(4):
                    idx = idx_v[s_in, pl.ds(bt * 128 + bb * _L, _L)]
                    for t in range(10):
                        val = plsc.load_gather(
                            tab_v.at[pl.ds(t * _L, _L)], [idx])
                        dst = outl_v if t < 5 else outf_v
                        dst[pl.ds((t % 5) * 4096 + bt * 1024 + s_in * 128 + bb * _L, _L)] = val
                return 0

            lax.fori_loop(0, 8, grp_body, 0)
            return 0

        lax.fori_loop(0, 8, row_body, 0)
        base = i * 131072 + wid * (4 * 1024)
        for t in range(5):
            pltpu.sync_copy(outl_v.at[pl.ds(t * 4096, 4096)],
                            out_l_hbm.at[pl.ds(t * _N + base, 4096)])
            pltpu.sync_copy(outf_v.at[pl.ds(t * 4096, 4096)],
                            out_f_hbm.at[pl.ds(t * _N + base, 4096)])
        return 0

    lax.fori_loop(0, _NCH, chunk, 0)


@jax.jit
def _run(xt, tab):
    mesh = plsc.VectorSubcoreMesh(core_axis_name="c", subcore_axis_name="s")
    f = pl.kernel(
        _body,
        out_type=(
            jax.ShapeDtypeStruct((5 * _N,), jnp.float32),
            jax.ShapeDtypeStruct((5 * _N,), jnp.float32),
        ),
        mesh=mesh,
        compiler_params=pltpu.CompilerParams(needs_layout_passes=False),
        scratch_types=[
            pltpu.VMEM((160,), jnp.float32),
            pltpu.VMEM((8, _BW), jnp.int32),
            pltpu.VMEM((5 * 4096,), jnp.float32),
            pltpu.VMEM((5 * 4096,), jnp.float32),
        ],
    )
    return f(xt, tab)


def kernel(x, W_learn, W_frozen):
    xt = x.T.astype(jnp.int32)
    tab = jnp.pad(
        jnp.concatenate([W_learn.T, W_frozen.T], axis=0), ((0, 0), (0, 6))
    ).reshape(-1)
    out_l, out_f = _run(xt, tab)

    def _assemble(flat):
        o5 = flat.reshape(5, 25, 128, 8, 128)
        return o5.transpose(2, 4, 1, 3, 0).reshape(_B, _S, 5)

    return (_assemble(out_l), _assemble(out_f))


# speedup vs baseline: 91.3435x; 1.2503x over previous
import numpy as _np

import jax
import jax.numpy as jnp
from jax import lax
from jax.experimental import pallas as pl
from jax.experimental.pallas import tpu as pltpu
from jax.experimental.pallas import tpu_sc as plsc

_B = 16384
_S = 200
_N = _B * _S
_NW = 32
_BW = _B // _NW
_NCH = _S // 8
_L = 16


def _body(xt_hbm, tab_hbm, out_l_hbm, out_f_hbm,
          tab_v, idx0, idx1, ol0, ol1, of0, of1, si0, si1, so0, so1):
    wid = lax.axis_index("s") * 2 + lax.axis_index("c")
    bcol = wid * _BW

    pltpu.sync_copy(tab_hbm, tab_v)

    idxb, olb, ofb = (idx0, idx1), (ol0, ol1), (of0, of1)
    sib, sob = (si0, si1), (so0, so1)

    def in_src(i):
        return xt_hbm.at[pl.ds(i * 8, 8), pl.ds(bcol, _BW)]

    def issue_in(i, b):
        pltpu.async_copy(in_src(i), idxb[b], sib[b])

    def wait_in(b):
        pltpu.make_async_copy(in_src(0), idxb[b], sib[b]).wait()

    def compute(b):
        idx_v, outl_v, outf_v = idxb[b], olb[b], ofb[b]

        def row_body(s_in, _):
            def grp_body(bb, _):
                for bt in range(4):
                    idx = idx_v[s_in, pl.ds(bt * 128 + bb * _L, _L)]
                    for t in range(10):
                        val = plsc.load_gather(
                            tab_v.at[pl.ds(t * _L, _L)], [idx])
                        dst = outl_v if t < 5 else outf_v
                        dst[pl.ds((t % 5) * 4096 + bt * 1024 + s_in * 128 + bb * _L, _L)] = val
                return 0

            lax.fori_loop(0, 8, grp_body, 0)
            return 0

        lax.fori_loop(0, 8, row_body, 0)

    def issue_out(i, b):
        base = i * 131072 + wid * (4 * 1024)
        for t in range(5):
            pltpu.async_copy(olb[b].at[pl.ds(t * 4096, 4096)],
                             out_l_hbm.at[pl.ds(t * _N + base, 4096)], sob[b])
            pltpu.async_copy(ofb[b].at[pl.ds(t * 4096, 4096)],
                             out_f_hbm.at[pl.ds(t * _N + base, 4096)], sob[b])

    def wait_out(b):
        for t in range(5):
            pltpu.make_async_copy(olb[b].at[pl.ds(0, 4096)],
                                  out_l_hbm.at[pl.ds(0, 4096)], sob[b]).wait()
            pltpu.make_async_copy(ofb[b].at[pl.ds(0, 4096)],
                                  out_f_hbm.at[pl.ds(0, 4096)], sob[b]).wait()

    issue_in(0, 0)
    issue_in(1, 1)

    def step(g, _):
        for b in range(2):
            wait_in(b)

            @pl.when(g >= 1)
            def _():
                wait_out(b)

            compute(b)
            issue_out(2 * g + b, b)
            if b == 0:
                issue_in(2 * g + 2, 0)
            else:
                @pl.when(g < 11)
                def _():
                    issue_in(2 * g + 3, 1)
        return 0

    lax.fori_loop(0, 12, step, 0)

    wait_in(0)
    wait_out(0)
    compute(0)
    issue_out(24, 0)
    wait_out(1)
    wait_out(0)


@jax.jit
def _run(xt, tab):
    mesh = plsc.VectorSubcoreMesh(core_axis_name="c", subcore_axis_name="s")
    f = pl.kernel(
        _body,
        out_type=(
            jax.ShapeDtypeStruct((5 * _N,), jnp.float32),
            jax.ShapeDtypeStruct((5 * _N,), jnp.float32),
        ),
        mesh=mesh,
        compiler_params=pltpu.CompilerParams(needs_layout_passes=False),
        scratch_types=[
            pltpu.VMEM((160,), jnp.float32),
            pltpu.VMEM((8, _BW), jnp.int32),
            pltpu.VMEM((8, _BW), jnp.int32),
            pltpu.VMEM((5 * 4096,), jnp.float32),
            pltpu.VMEM((5 * 4096,), jnp.float32),
            pltpu.VMEM((5 * 4096,), jnp.float32),
            pltpu.VMEM((5 * 4096,), jnp.float32),
            pltpu.SemaphoreType.DMA,
            pltpu.SemaphoreType.DMA,
            pltpu.SemaphoreType.DMA,
            pltpu.SemaphoreType.DMA,
        ],
    )
    return f(xt, tab)


def kernel(x, W_learn, W_frozen):
    xt = x.T.astype(jnp.int32)
    tab = jnp.pad(
        jnp.concatenate([W_learn.T, W_frozen.T], axis=0), ((0, 0), (0, 6))
    ).reshape(-1)
    out_l, out_f = _run(xt, tab)

    def _assemble(flat):
        o5 = flat.reshape(5, 25, 128, 8, 128)
        return o5.transpose(2, 4, 1, 3, 0).reshape(_B, _S, 5)

    return (_assemble(out_l), _assemble(out_f))


# speedup vs baseline: 143.8639x; 1.5750x over previous
import numpy as _np

import jax
import jax.numpy as jnp
from jax import lax
from jax.experimental import pallas as pl
from jax.experimental.pallas import tpu as pltpu
from jax.experimental.pallas import tpu_sc as plsc

_B = 16384
_S = 200
_N = _B * _S
_NW = 32
_BW = _B // _NW
_NCH = _S // 8
_L = 16


def _body(xt_hbm, tab_hbm, out_l_hbm, out_f_hbm,
          tab_v, idx0, idx1, ol0, ol1, of0, of1, si0, si1, so0, so1):
    wid = lax.axis_index("s") * 2 + lax.axis_index("c")
    bcol = wid * _BW

    pltpu.sync_copy(tab_hbm, tab_v)

    idxb, olb, ofb = (idx0, idx1), (ol0, ol1), (of0, of1)
    sib, sob = (si0, si1), (so0, so1)

    def in_src(i):
        return xt_hbm.at[pl.ds(i * 8, 8), pl.ds(bcol, _BW)]

    def issue_in(i, b):
        pltpu.async_copy(in_src(i), idxb[b], sib[b])

    def wait_in(b):
        pltpu.make_async_copy(in_src(0), idxb[b], sib[b]).wait()

    def compute(b):
        idx_v, outl_v, outf_v = idxb[b], olb[b], ofb[b]

        @plsc.parallel_loop(0, 8)
        def row_body(s_in):
            srow = s_in * 128
            for bt in range(4):
                for bb in range(8):
                    idx = idx_v[s_in, pl.ds(bt * 128 + bb * _L, _L)]
                    for t in range(10):
                        val = plsc.load_gather(
                            tab_v.at[pl.ds(t * _L, _L)], [idx])
                        dst = outl_v if t < 5 else outf_v
                        dst[pl.ds(srow + (t % 5) * 4096 + bt * 1024 + bb * _L, _L)] = val

    def issue_out(i, b):
        base = i * 131072 + wid * (4 * 1024)
        for t in range(5):
            pltpu.async_copy(olb[b].at[pl.ds(t * 4096, 4096)],
                             out_l_hbm.at[pl.ds(t * _N + base, 4096)], sob[b])
            pltpu.async_copy(ofb[b].at[pl.ds(t * 4096, 4096)],
                             out_f_hbm.at[pl.ds(t * _N + base, 4096)], sob[b])

    def wait_out(b):
        for t in range(5):
            pltpu.make_async_copy(olb[b].at[pl.ds(0, 4096)],
                                  out_l_hbm.at[pl.ds(0, 4096)], sob[b]).wait()
            pltpu.make_async_copy(ofb[b].at[pl.ds(0, 4096)],
                                  out_f_hbm.at[pl.ds(0, 4096)], sob[b]).wait()

    issue_in(0, 0)
    issue_in(1, 1)

    def step(g, _):
        wait_in(0)

        @pl.when(g >= 1)
        def _():
            wait_out(0)

        compute(0)
        issue_out(2 * g, 0)

        @pl.when(g < 12)
        def _():
            issue_in(2 * g + 2, 0)

        @pl.when(g < 12)
        def _():
            wait_in(1)

            @pl.when(g >= 1)
            def _():
                wait_out(1)

            compute(1)
            issue_out(2 * g + 1, 1)

            @pl.when(g < 11)
            def _():
                issue_in(2 * g + 3, 1)

        return 0

    lax.fori_loop(0, 13, step, 0)

    wait_out(1)
    wait_out(0)


@jax.jit
def _run(xt, tab):
    mesh = plsc.VectorSubcoreMesh(core_axis_name="c", subcore_axis_name="s")
    f = pl.kernel(
        _body,
        out_type=(
            jax.ShapeDtypeStruct((5 * _N,), jnp.float32),
            jax.ShapeDtypeStruct((5 * _N,), jnp.float32),
        ),
        mesh=mesh,
        compiler_params=pltpu.CompilerParams(needs_layout_passes=False),
        scratch_types=[
            pltpu.VMEM((160,), jnp.float32),
            pltpu.VMEM((8, _BW), jnp.int32),
            pltpu.VMEM((8, _BW), jnp.int32),
            pltpu.VMEM((5 * 4096,), jnp.float32),
            pltpu.VMEM((5 * 4096,), jnp.float32),
            pltpu.VMEM((5 * 4096,), jnp.float32),
            pltpu.VMEM((5 * 4096,), jnp.float32),
            pltpu.SemaphoreType.DMA,
            pltpu.SemaphoreType.DMA,
            pltpu.SemaphoreType.DMA,
            pltpu.SemaphoreType.DMA,
        ],
    )
    return f(xt, tab)


def kernel(x, W_learn, W_frozen):
    xt = x.T.astype(jnp.int32)
    tab = jnp.pad(
        jnp.concatenate([W_learn.T, W_frozen.T], axis=0), ((0, 0), (0, 6))
    ).reshape(-1)
    out_l, out_f = _run(xt, tab)

    def _assemble(flat):
        o5 = flat.reshape(5, 25, 128, 8, 128)
        return o5.transpose(2, 4, 1, 3, 0).reshape(_B, _S, 5)

    return (_assemble(out_l), _assemble(out_f))
